# baseline (speedup 1.0000x reference)
import functools

import jax
import jax.numpy as jnp
from jax.experimental import pallas as pl
from jax.experimental.pallas import tpu as pltpu

EMB_DIMS = (768, 1024)
LATENT = 64
NUM_CB = 4
CB_SIZE = 64
BETA = 0.25
BATCH = 16384
BT = 2048


def _dot(a, b):
    return jnp.dot(a.astype(jnp.bfloat16), b.astype(jnp.bfloat16),
                   preferred_element_type=jnp.float32)


def _mlp3(x, w0, b0, w1, b1, w2, b2):
    h = jnp.maximum(_dot(x, w0) + b0, 0.0)
    h = jnp.maximum(_dot(h, w1) + b1, 0.0)
    return _dot(h, w2) + b2


def _vq_stage(state, c, cbs_ref, cbT_ref, hi_ref, mid_ref, lo_ref, iiota):
    residual, z_q, rsq, rq_sum, id_cols = state
    cb = cbs_ref[c]
    csq = jnp.sum(cb * cb, axis=1)[None, :]
    d = rsq - 2.0 * jnp.dot(residual.astype(jnp.bfloat16), cbT_ref[c],
                            preferred_element_type=jnp.float32) + csq
    dmin = jnp.min(d, axis=1, keepdims=True)
    idx = jnp.min(jnp.where(d == dmin, iiota, CB_SIZE), axis=1)[:, None]
    onehot = (iiota == idx).astype(jnp.bfloat16)
    q = (jnp.dot(onehot, hi_ref[c], preferred_element_type=jnp.float32)
         + jnp.dot(onehot, mid_ref[c], preferred_element_type=jnp.float32)
         + jnp.dot(onehot, lo_ref[c], preferred_element_type=jnp.float32))
    diff = residual - q
    rsq = jnp.sum(diff * diff, axis=1, keepdims=True)
    rq_sum = rq_sum + jnp.sum(rsq)
    return (diff, z_q + q, rsq, rq_sum, id_cols + [idx[:, 0]])


def _rqvae_pair(x0, enc0, dec0, vq0, x1, enc1, dec1, vq1):
    iiota = jax.lax.broadcasted_iota(jnp.int32, (BT, CB_SIZE), 1)
    z0 = _mlp3(x0, *enc0)
    z1 = _mlp3(x1, *enc1)
    s0 = (z0, jnp.zeros_like(z0),
          jnp.sum(z0 * z0, axis=1, keepdims=True), jnp.float32(0.0), [])
    s1 = (z1, jnp.zeros_like(z1),
          jnp.sum(z1 * z1, axis=1, keepdims=True), jnp.float32(0.0), [])
    for c in range(NUM_CB):
        s0 = _vq_stage(s0, c, *vq0, iiota)
        s1 = _vq_stage(s1, c, *vq1, iiota)
    x_hat0 = _mlp3(s0[1], *dec0)
    x_hat1 = _mlp3(s1[1], *dec1)
    e0 = x_hat0 - x0
    e1 = x_hat1 - x1
    out0 = (jnp.stack(s0[4], axis=1), jnp.sum(e0 * e0), s0[3])
    out1 = (jnp.stack(s1[4], axis=1), jnp.sum(e1 * e1), s1[3])
    return out0, out1


def _body(f0_ref, f1_ref,
          e0w0, e0b0, e0w1, e0b1, e0w2, e0b2,
          d0w0, d0b0, d0w1, d0b1, d0w2, d0b2,
          cbs0, cbT0, hi0, mid0, lo0,
          e1w0, e1b0, e1w1, e1b1, e1w2, e1b2,
          d1w0, d1b0, d1w1, d1b1, d1w2, d1b2,
          cbs1, cbT1, hi1, mid1, lo1,
          ids0_ref, ids1_ref, loss_ref):
    enc0 = (e0w0[...], e0b0[...], e0w1[...], e0b1[...], e0w2[...], e0b2[...])
    dec0 = (d0w0[...], d0b0[...], d0w1[...], d0b1[...], d0w2[...], d0b2[...])
    enc1 = (e1w0[...], e1b0[...], e1w1[...], e1b1[...], e1w2[...], e1b2[...])
    dec1 = (d1w0[...], d1b0[...], d1w1[...], d1b1[...], d1w2[...], d1b2[...])
    (ids0, recon0, rq0), (ids1, recon1, rq1) = _rqvae_pair(
        f0_ref[...], enc0, dec0, (cbs0, cbT0, hi0, mid0, lo0),
        f1_ref[...], enc1, dec1, (cbs1, cbT1, hi1, mid1, lo1))
    ids0_ref[...] = ids0
    ids1_ref[...] = ids1

    loss_ref[0, 0, 0] = recon0
    loss_ref[0, 0, 1] = rq0
    loss_ref[0, 0, 2] = recon1
    loss_ref[0, 0, 3] = rq1


@functools.partial(jax.jit, static_argnames=("interpret",))
def _run(feat0, feat1, weights0, weights1, vq0, vq1, interpret=False):
    grid = (BATCH // BT,)

    def full(a):
        return pl.BlockSpec(a.shape, lambda i: (0,) * a.ndim)

    in_specs = (
        [pl.BlockSpec((BT, EMB_DIMS[0]), lambda i: (i, 0)),
         pl.BlockSpec((BT, EMB_DIMS[1]), lambda i: (i, 0))]
        + [full(w) for w in weights0] + [full(a) for a in vq0]
        + [full(w) for w in weights1] + [full(a) for a in vq1]
    )
    out_specs = (
        pl.BlockSpec((BT, NUM_CB), lambda i: (i, 0)),
        pl.BlockSpec((BT, NUM_CB), lambda i: (i, 0)),
        pl.BlockSpec((1, 1, 4), lambda i: (i, 0, 0), memory_space=pltpu.SMEM),
    )
    out_shapes = (
        jax.ShapeDtypeStruct((BATCH, NUM_CB), jnp.int32),
        jax.ShapeDtypeStruct((BATCH, NUM_CB), jnp.int32),
        jax.ShapeDtypeStruct((grid[0], 1, 4), jnp.float32),
    )
    return pl.pallas_call(
        _body,
        grid=grid,
        in_specs=in_specs,
        out_specs=out_specs,
        out_shape=out_shapes,
        interpret=interpret,
        compiler_params=pltpu.CompilerParams(
            dimension_semantics=("parallel",)),
    )(feat0, feat1, *weights0, *vq0, *weights1, *vq1)


def _split_codebooks(cbs):
    barrier = jax.lax.optimization_barrier
    hi = barrier(cbs.astype(jnp.bfloat16))
    r1 = cbs - barrier(hi.astype(jnp.float32))
    mid = barrier(r1.astype(jnp.bfloat16))
    lo = (r1 - barrier(mid.astype(jnp.float32))).astype(jnp.bfloat16)
    cbT = jnp.transpose(hi, (0, 2, 1))
    return (cbs, cbT, hi, mid, lo)


def kernel(feat0, feat1,
           m0_enc_w0, m0_enc_b0, m0_enc_w1, m0_enc_b1, m0_enc_w2, m0_enc_b2,
           m0_dec_w0, m0_dec_b0, m0_dec_w1, m0_dec_b1, m0_dec_w2, m0_dec_b2,
           m0_codebooks,
           m1_enc_w0, m1_enc_b0, m1_enc_w1, m1_enc_b1, m1_enc_w2, m1_enc_b2,
           m1_dec_w0, m1_dec_b0, m1_dec_w1, m1_dec_b1, m1_dec_w2, m1_dec_b2,
           m1_codebooks):
    r = lambda b: b.reshape(1, -1)
    w = lambda m: m.astype(jnp.bfloat16)
    weights0 = (w(m0_enc_w0), r(m0_enc_b0), w(m0_enc_w1), r(m0_enc_b1),
                w(m0_enc_w2), r(m0_enc_b2),
                w(m0_dec_w0), r(m0_dec_b0), w(m0_dec_w1), r(m0_dec_b1),
                w(m0_dec_w2), r(m0_dec_b2))
    weights1 = (w(m1_enc_w0), r(m1_enc_b0), w(m1_enc_w1), r(m1_enc_b1),
                w(m1_enc_w2), r(m1_enc_b2),
                w(m1_dec_w0), r(m1_dec_b0), w(m1_dec_w1), r(m1_dec_b1),
                w(m1_dec_w2), r(m1_dec_b2))
    vq0 = _split_codebooks(m0_codebooks)
    vq1 = _split_codebooks(m1_codebooks)
    ids0, ids1, part = _run(feat0, feat1, weights0, weights1, vq0, vq1)
    sums = jnp.sum(part[:, 0, :], axis=0)
    n_rq = jnp.float32(BATCH * LATENT)
    total0 = sums[0] / (BATCH * EMB_DIMS[0]) + (1.0 + BETA) * sums[1] / n_rq
    total1 = sums[2] / (BATCH * EMB_DIMS[1]) + (1.0 + BETA) * sums[3] / n_rq
    return (ids0, ids1, total0, total1)


# speedup vs baseline: 1.1900x; 1.1900x over previous
import functools

import jax
import jax.numpy as jnp
from jax.experimental import pallas as pl
from jax.experimental.pallas import tpu as pltpu

EMB_DIMS = (768, 1024)
LATENT = 64
NUM_CB = 4
CB_SIZE = 64
BETA = 0.25
BATCH = 16384
BT = 2048


def _dot(a, b):
    return jnp.dot(a.astype(jnp.bfloat16), b.astype(jnp.bfloat16),
                   preferred_element_type=jnp.float32)


def _mlp3(x, w0, b0, w1, b1, w2, b2):
    h = jnp.maximum(_dot(x, w0) + b0, 0.0)
    h = jnp.maximum(_dot(h, w1) + b1, 0.0)
    return _dot(h, w2) + b2


def _vq_stage(state, c, cbs_ref, cbT_ref, hi_ref, mid_ref, lo_ref, iiota):
    residual, z_q, rsq, rq_sum, id_cols = state
    cb = cbs_ref[c]
    csq = jnp.sum(cb * cb, axis=1)[None, :]
    d = rsq - 2.0 * jnp.dot(residual.astype(jnp.bfloat16), cbT_ref[c],
                            preferred_element_type=jnp.float32) + csq
    idx = jnp.argmin(d, axis=1)[:, None]
    onehot = (iiota == idx).astype(jnp.bfloat16)
    q = (jnp.dot(onehot, hi_ref[c], preferred_element_type=jnp.float32)
         + jnp.dot(onehot, mid_ref[c], preferred_element_type=jnp.float32)
         + jnp.dot(onehot, lo_ref[c], preferred_element_type=jnp.float32))
    diff = residual - q
    rsq = jnp.sum(diff * diff, axis=1, keepdims=True)
    rq_sum = rq_sum + jnp.sum(rsq)
    return (diff, z_q + q, rsq, rq_sum, id_cols + [idx[:, 0]])


def _rqvae_pair(x0, enc0, dec0, vq0, x1, enc1, dec1, vq1):
    iiota = jax.lax.broadcasted_iota(jnp.int32, (BT, CB_SIZE), 1)
    z0 = _mlp3(x0, *enc0)
    z1 = _mlp3(x1, *enc1)
    s0 = (z0, jnp.zeros_like(z0),
          jnp.sum(z0 * z0, axis=1, keepdims=True), jnp.float32(0.0), [])
    s1 = (z1, jnp.zeros_like(z1),
          jnp.sum(z1 * z1, axis=1, keepdims=True), jnp.float32(0.0), [])
    for c in range(NUM_CB):
        s0 = _vq_stage(s0, c, *vq0, iiota)
        s1 = _vq_stage(s1, c, *vq1, iiota)
    x_hat0 = _mlp3(s0[1], *dec0)
    x_hat1 = _mlp3(s1[1], *dec1)
    e0 = x_hat0 - x0
    e1 = x_hat1 - x1
    out0 = (jnp.stack(s0[4], axis=1), jnp.sum(e0 * e0), s0[3])
    out1 = (jnp.stack(s1[4], axis=1), jnp.sum(e1 * e1), s1[3])
    return out0, out1


def _body(f0_ref, f1_ref,
          e0w0, e0b0, e0w1, e0b1, e0w2, e0b2,
          d0w0, d0b0, d0w1, d0b1, d0w2, d0b2,
          cbs0, cbT0, hi0, mid0, lo0,
          e1w0, e1b0, e1w1, e1b1, e1w2, e1b2,
          d1w0, d1b0, d1w1, d1b1, d1w2, d1b2,
          cbs1, cbT1, hi1, mid1, lo1,
          ids0_ref, ids1_ref, loss_ref):
    enc0 = (e0w0[...], e0b0[...], e0w1[...], e0b1[...], e0w2[...], e0b2[...])
    dec0 = (d0w0[...], d0b0[...], d0w1[...], d0b1[...], d0w2[...], d0b2[...])
    enc1 = (e1w0[...], e1b0[...], e1w1[...], e1b1[...], e1w2[...], e1b2[...])
    dec1 = (d1w0[...], d1b0[...], d1w1[...], d1b1[...], d1w2[...], d1b2[...])
    (ids0, recon0, rq0), (ids1, recon1, rq1) = _rqvae_pair(
        f0_ref[...], enc0, dec0, (cbs0, cbT0, hi0, mid0, lo0),
        f1_ref[...], enc1, dec1, (cbs1, cbT1, hi1, mid1, lo1))
    ids0_ref[...] = ids0
    ids1_ref[...] = ids1

    loss_ref[0, 0, 0] = recon0
    loss_ref[0, 0, 1] = rq0
    loss_ref[0, 0, 2] = recon1
    loss_ref[0, 0, 3] = rq1


@functools.partial(jax.jit, static_argnames=("interpret",))
def _run(feat0, feat1, weights0, weights1, vq0, vq1, interpret=False):
    grid = (BATCH // BT,)

    def full(a):
        return pl.BlockSpec(a.shape, lambda i: (0,) * a.ndim)

    in_specs = (
        [pl.BlockSpec((BT, EMB_DIMS[0]), lambda i: (i, 0)),
         pl.BlockSpec((BT, EMB_DIMS[1]), lambda i: (i, 0))]
        + [full(w) for w in weights0] + [full(a) for a in vq0]
        + [full(w) for w in weights1] + [full(a) for a in vq1]
    )
    out_specs = (
        pl.BlockSpec((BT, NUM_CB), lambda i: (i, 0)),
        pl.BlockSpec((BT, NUM_CB), lambda i: (i, 0)),
        pl.BlockSpec((1, 1, 4), lambda i: (i, 0, 0), memory_space=pltpu.SMEM),
    )
    out_shapes = (
        jax.ShapeDtypeStruct((BATCH, NUM_CB), jnp.int32),
        jax.ShapeDtypeStruct((BATCH, NUM_CB), jnp.int32),
        jax.ShapeDtypeStruct((grid[0], 1, 4), jnp.float32),
    )
    return pl.pallas_call(
        _body,
        grid=grid,
        in_specs=in_specs,
        out_specs=out_specs,
        out_shape=out_shapes,
        interpret=interpret,
        compiler_params=pltpu.CompilerParams(
            dimension_semantics=("parallel",)),
    )(feat0, feat1, *weights0, *vq0, *weights1, *vq1)


def _split_codebooks(cbs):
    barrier = jax.lax.optimization_barrier
    hi = barrier(cbs.astype(jnp.bfloat16))
    r1 = cbs - barrier(hi.astype(jnp.float32))
    mid = barrier(r1.astype(jnp.bfloat16))
    lo = (r1 - barrier(mid.astype(jnp.float32))).astype(jnp.bfloat16)
    cbT = jnp.transpose(hi, (0, 2, 1))
    return (cbs, cbT, hi, mid, lo)


def kernel(feat0, feat1,
           m0_enc_w0, m0_enc_b0, m0_enc_w1, m0_enc_b1, m0_enc_w2, m0_enc_b2,
           m0_dec_w0, m0_dec_b0, m0_dec_w1, m0_dec_b1, m0_dec_w2, m0_dec_b2,
           m0_codebooks,
           m1_enc_w0, m1_enc_b0, m1_enc_w1, m1_enc_b1, m1_enc_w2, m1_enc_b2,
           m1_dec_w0, m1_dec_b0, m1_dec_w1, m1_dec_b1, m1_dec_w2, m1_dec_b2,
           m1_codebooks):
    r = lambda b: b.reshape(1, -1)
    w = lambda m: m.astype(jnp.bfloat16)
    weights0 = (w(m0_enc_w0), r(m0_enc_b0), w(m0_enc_w1), r(m0_enc_b1),
                w(m0_enc_w2), r(m0_enc_b2),
                w(m0_dec_w0), r(m0_dec_b0), w(m0_dec_w1), r(m0_dec_b1),
                w(m0_dec_w2), r(m0_dec_b2))
    weights1 = (w(m1_enc_w0), r(m1_enc_b0), w(m1_enc_w1), r(m1_enc_b1),
                w(m1_enc_w2), r(m1_enc_b2),
                w(m1_dec_w0), r(m1_dec_b0), w(m1_dec_w1), r(m1_dec_b1),
                w(m1_dec_w2), r(m1_dec_b2))
    vq0 = _split_codebooks(m0_codebooks)
    vq1 = _split_codebooks(m1_codebooks)
    ids0, ids1, part = _run(feat0, feat1, weights0, weights1, vq0, vq1)
    sums = jnp.sum(part[:, 0, :], axis=0)
    n_rq = jnp.float32(BATCH * LATENT)
    total0 = sums[0] / (BATCH * EMB_DIMS[0]) + (1.0 + BETA) * sums[1] / n_rq
    total1 = sums[2] / (BATCH * EMB_DIMS[1]) + (1.0 + BETA) * sums[3] / n_rq
    return (ids0, ids1, total0, total1)
